# baseline (speedup 1.0000x reference)
import functools

import jax
import jax.numpy as jnp
import numpy as np
from jax import lax
from jax.experimental import pallas as pl
from jax.experimental.pallas import tpu as pltpu
from jax.experimental.pallas import tpu_sc as plsc

_VOCAB = 100000
_D = 128
_SEQ = 2048
_BATCH = 4
_NC = 2
_NS = 16
_NW = _NC * _NS
_ROWS = (_BATCH * _SEQ) // _NW


def _pe_table() -> np.ndarray:
    pos = np.arange(_SEQ, dtype=np.float32)[:, None]
    div = np.exp(np.arange(0, _D, 2, dtype=np.float32) * (-np.log(10000.0) / _D))
    pe = np.zeros((_SEQ, _D), dtype=np.float32)
    pe[:, 0::2] = np.sin(pos * div)
    pe[:, 1::2] = np.cos(pos * div)
    return pe


_PE = _pe_table()


_G = 4
_C = _ROWS // _G


def _sc_body(x_hbm, pe_hbm, table_hbm, out_hbm,
             idx_v, gb0, gb1, rb0, rb1, pe_v, sg0, sg1, so0, so1):
    wid = lax.axis_index("s") * _NC + lax.axis_index("c")
    base = wid * _ROWS
    pltpu.sync_copy(x_hbm.at[pl.ds(base, _ROWS)], idx_v)
    gbufs, rbufs = (gb0, gb1), (rb0, rb1)
    sgs, sos = (sg0, sg1), (so0, so1)
    gathers = [None] * _G
    scatters = [None] * _G
    gathers[0] = pltpu.async_copy(
        table_hbm.at[idx_v.at[pl.ds(0, _C)]], gb0, sg0)
    pe_base = lax.rem(base, _SEQ)
    pltpu.sync_copy(pe_hbm.at[pl.ds(pe_base, _ROWS)], pe_v)
    gathers[1] = pltpu.async_copy(
        table_hbm.at[idx_v.at[pl.ds(_C, _C)]], gb1, sg1)
    for g in range(_G):
        b = g % 2
        gathers[g].wait()
        if g >= 2:
            scatters[g - 2].wait()
        gb, rb = gbufs[b], rbufs[b]
        off = g * _C

        @plsc.parallel_loop(0, _C, unroll=4)
        def add_row(i, gb=gb, rb=rb, off=off):
            for c in range(_D // 16):
                sl = pl.ds(c * 16, 16)
                rb[i, sl] = gb[i, sl] + pe_v[off + i, sl]

        if g + 2 < _G:
            gathers[g + 2] = pltpu.async_copy(
                table_hbm.at[idx_v.at[pl.ds((g + 2) * _C, _C)]],
                gbufs[b], sgs[b])
        scatters[g] = pltpu.async_copy(
            rb, out_hbm.at[pl.ds(base + off, _C)], sos[b])
    scatters[_G - 2].wait()
    scatters[_G - 1].wait()


@functools.partial(jax.jit, static_argnames=())
def _run(x_flat, pe, table):
    mesh = plsc.VectorSubcoreMesh(core_axis_name="c", subcore_axis_name="s")
    f = pl.kernel(
        _sc_body,
        mesh=mesh,
        out_type=jax.ShapeDtypeStruct((_BATCH * _SEQ, _D), jnp.float32),
        scratch_types=[
            pltpu.VMEM((_ROWS,), jnp.int32),
            pltpu.VMEM((_C, _D), jnp.float32),
            pltpu.VMEM((_C, _D), jnp.float32),
            pltpu.VMEM((_C, _D), jnp.float32),
            pltpu.VMEM((_C, _D), jnp.float32),
            pltpu.VMEM((_ROWS, _D), jnp.float32),
            pltpu.SemaphoreType.DMA,
            pltpu.SemaphoreType.DMA,
            pltpu.SemaphoreType.DMA,
            pltpu.SemaphoreType.DMA,
        ],
    )
    return f(x_flat, pe, table)


def kernel(x, table):
    out = _run(x.reshape(-1), _PE, table)
    return out.reshape(_BATCH, _SEQ, _D)


# speedup vs baseline: 1.0105x; 1.0105x over previous
import functools

import jax
import jax.numpy as jnp
import numpy as np
from jax import lax
from jax.experimental import pallas as pl
from jax.experimental.pallas import tpu as pltpu
from jax.experimental.pallas import tpu_sc as plsc

_VOCAB = 100000
_D = 128
_SEQ = 2048
_BATCH = 4
_NC = 2
_NS = 16
_NW = _NC * _NS
_ROWS = (_BATCH * _SEQ) // _NW


def _pe_table() -> np.ndarray:
    pos = np.arange(_SEQ, dtype=np.float32)[:, None]
    div = np.exp(np.arange(0, _D, 2, dtype=np.float32) * (-np.log(10000.0) / _D))
    pe = np.zeros((_SEQ, _D), dtype=np.float32)
    pe[:, 0::2] = np.sin(pos * div)
    pe[:, 1::2] = np.cos(pos * div)
    return pe


_PE = _pe_table()


_G = 4
_C = _ROWS // _G


def _sc_body(x_hbm, pe_hbm, table_hbm, out_hbm,
             idx_v, gb0, gb1, rb0, rb1, pe_v, sg0, sg1, so0, so1):
    wid = lax.axis_index("s") * _NC + lax.axis_index("c")
    base = wid * _ROWS
    batch = wid // (_SEQ // _ROWS)
    col = lax.rem(base, _SEQ)
    pltpu.sync_copy(x_hbm.at[batch, pl.ds(col, _ROWS)], idx_v)
    gbufs, rbufs = (gb0, gb1), (rb0, rb1)
    sgs, sos = (sg0, sg1), (so0, so1)
    gathers = [None] * _G
    scatters = [None] * _G
    gathers[0] = pltpu.async_copy(
        table_hbm.at[idx_v.at[pl.ds(0, _C)]], gb0, sg0)
    pe_base = lax.rem(base, _SEQ)
    pltpu.sync_copy(pe_hbm.at[pl.ds(pe_base, _ROWS)], pe_v)
    gathers[1] = pltpu.async_copy(
        table_hbm.at[idx_v.at[pl.ds(_C, _C)]], gb1, sg1)
    for g in range(_G):
        b = g % 2
        gathers[g].wait()
        if g >= 2:
            scatters[g - 2].wait()
        gb, rb = gbufs[b], rbufs[b]
        off = g * _C

        @plsc.parallel_loop(0, _C, unroll=4)
        def add_row(i, gb=gb, rb=rb, off=off):
            for c in range(_D // 16):
                sl = pl.ds(c * 16, 16)
                rb[i, sl] = gb[i, sl] + pe_v[off + i, sl]

        if g + 2 < _G:
            gathers[g + 2] = pltpu.async_copy(
                table_hbm.at[idx_v.at[pl.ds((g + 2) * _C, _C)]],
                gbufs[b], sgs[b])
        scatters[g] = pltpu.async_copy(
            rb, out_hbm.at[pl.ds(base + off, _C)], sos[b])
    scatters[_G - 2].wait()
    scatters[_G - 1].wait()


@functools.partial(jax.jit, static_argnames=())
def _run(x2d, pe, table):
    mesh = plsc.VectorSubcoreMesh(core_axis_name="c", subcore_axis_name="s")
    f = pl.kernel(
        _sc_body,
        mesh=mesh,
        out_type=jax.ShapeDtypeStruct((_BATCH * _SEQ, _D), jnp.float32),
        scratch_types=[
            pltpu.VMEM((_ROWS,), jnp.int32),
            pltpu.VMEM((_C, _D), jnp.float32),
            pltpu.VMEM((_C, _D), jnp.float32),
            pltpu.VMEM((_C, _D), jnp.float32),
            pltpu.VMEM((_C, _D), jnp.float32),
            pltpu.VMEM((_ROWS, _D), jnp.float32),
            pltpu.SemaphoreType.DMA,
            pltpu.SemaphoreType.DMA,
            pltpu.SemaphoreType.DMA,
            pltpu.SemaphoreType.DMA,
        ],
    )
    return f(x2d, pe, table)


def kernel(x, table):
    out = _run(x, _PE, table)
    return out.reshape(_BATCH, _SEQ, _D)
